# baseline (speedup 1.0000x reference)
import functools

import jax
import jax.numpy as jnp
from jax import lax
from jax.experimental import pallas as pl
from jax.experimental.pallas import tpu as pltpu
from jax.experimental.pallas import tpu_sc as plsc

_BATCH = 16384
_D = 128
_NW = 32
_BPW = _BATCH // _NW
_CHUNK = 128
_NCH = _BPW // _CHUNK
_NRB = 3


def _sc_add_gather(x, ind2, mapping, weight):
    mesh = plsc.VectorSubcoreMesh(core_axis_name="c", subcore_axis_name="s")

    @functools.partial(
        pl.kernel,
        mesh=mesh,
        out_type=jax.ShapeDtypeStruct((_BATCH, _D), jnp.float32),
        scratch_types=[
            pltpu.VMEM((_BPW,), jnp.int32),
            pltpu.VMEM((_NCH, _CHUNK), jnp.int32),
            pltpu.VMEM((_NRB, _CHUNK, _D), jnp.float32),
            pltpu.VMEM((_NCH, _CHUNK, _D), jnp.float32),
        ]
        + [pltpu.SemaphoreType.DMA] * _NCH
        + [pltpu.SemaphoreType.DMA] * _NRB
        + [pltpu.SemaphoreType.DMA] * _NCH
        + [pltpu.SemaphoreType.DMA]
        + [pltpu.SemaphoreType.DMA],
    )
    def k(x_hbm, ind_hbm, map_hbm, w_hbm, out_hbm, idx_v, cid_v, rows_v, xb_v,
          *sems):
        csem = sems[0:_NCH]
        gsem = sems[_NCH:_NCH + _NRB]
        xsem = sems[_NCH + _NRB:2 * _NCH + _NRB]
        osem = sems[2 * _NCH + _NRB]
        isem = sems[2 * _NCH + _NRB + 1]
        wid = lax.axis_index("s") * 2 + lax.axis_index("c")
        base = wid * _BPW
        ind_cp = pltpu.async_copy(ind_hbm.at[pl.ds(base, _BPW)], idx_v, isem)
        x_cps = [
            pltpu.async_copy(
                x_hbm.at[pl.ds(base + j * _CHUNK, _CHUNK)], xb_v.at[j], xsem[j])
            for j in range(_NCH)
        ]
        ind_cp.wait()
        cid_cps = [
            pltpu.async_copy(
                map_hbm.at[idx_v.at[pl.ds(j * _CHUNK, _CHUNK)]],
                cid_v.at[j], csem[j])
            for j in range(_NCH)
        ]

        g_cps = [None] * _NCH

        def fire_rows(j):
            cid_cps[j].wait()
            g_cps[j] = pltpu.async_copy(
                w_hbm.at[cid_v.at[j]], rows_v.at[j % _NRB], gsem[j % _NRB])

        for j in range(_NRB):
            fire_rows(j)

        out_cps = []
        for j in range(_NCH):
            rb = j % _NRB
            g_cps[j].wait()
            x_cps[j].wait()

            def body(r4, carry):
                for dr in range(4):
                    r = r4 * 4 + dr
                    for gi in range(_D // 16):
                        s = pl.ds(gi * 16, 16)
                        xb_v[j, r, s] = xb_v[j, r, s] + rows_v[rb, r, s]
                return carry

            lax.fori_loop(0, _CHUNK // 4, body, 0)
            out_cps.append(pltpu.async_copy(
                xb_v.at[j], out_hbm.at[pl.ds(base + j * _CHUNK, _CHUNK)], osem))
            if j + _NRB < _NCH:
                fire_rows(j + _NRB)
        for cp in out_cps:
            cp.wait()

    return k(x, ind2, mapping, weight)


def kernel(x, ind, mapping, weight):
    return _sc_add_gather(x, ind.astype(jnp.int32), mapping.astype(jnp.int32), weight)


# speedup vs baseline: 1.0069x; 1.0069x over previous
import functools

import jax
import jax.numpy as jnp
from jax import lax
from jax.experimental import pallas as pl
from jax.experimental.pallas import tpu as pltpu
from jax.experimental.pallas import tpu_sc as plsc

_BATCH = 16384
_D = 128
_NW = 32
_BPW = _BATCH // _NW
_CHUNK = 128
_NCH = _BPW // _CHUNK
_NRB = 3


def _sc_add_gather(x, ind2, mapping, weight):
    mesh = plsc.VectorSubcoreMesh(core_axis_name="c", subcore_axis_name="s")

    @functools.partial(
        pl.kernel,
        mesh=mesh,
        out_type=jax.ShapeDtypeStruct((_BATCH, _D), jnp.float32),
        scratch_types=[
            pltpu.VMEM((_BPW,), jnp.int32),
            pltpu.VMEM((_NCH, _CHUNK), jnp.int32),
            pltpu.VMEM((_NRB, _CHUNK, _D), jnp.float32),
            pltpu.VMEM((_NCH, _CHUNK, _D), jnp.float32),
        ]
        + [pltpu.SemaphoreType.DMA] * _NCH
        + [pltpu.SemaphoreType.DMA] * _NRB
        + [pltpu.SemaphoreType.DMA] * _NCH
        + [pltpu.SemaphoreType.DMA]
        + [pltpu.SemaphoreType.DMA],
    )
    def k(x_hbm, ind_hbm, map_hbm, w_hbm, out_hbm, idx_v, cid_v, rows_v, xb_v,
          *sems):
        csem = sems[0:_NCH]
        gsem = sems[_NCH:_NCH + _NRB]
        xsem = sems[_NCH + _NRB:2 * _NCH + _NRB]
        osem = sems[2 * _NCH + _NRB]
        isem = sems[2 * _NCH + _NRB + 1]
        wid = lax.axis_index("s") * 2 + lax.axis_index("c")
        base = wid * _BPW
        ind_cp = pltpu.async_copy(ind_hbm.at[pl.ds(base, _BPW)], idx_v, isem)
        x_cps = [
            pltpu.async_copy(
                x_hbm.at[pl.ds(base + j * _CHUNK, _CHUNK)], xb_v.at[j], xsem[j])
            for j in range(_NCH)
        ]
        ind_cp.wait()
        cid_cps = [
            pltpu.async_copy(
                map_hbm.at[idx_v.at[pl.ds(j * _CHUNK, _CHUNK)]],
                cid_v.at[j], csem[j])
            for j in range(_NCH)
        ]

        g_cps = [None] * _NCH

        def fire_rows(j):
            cid_cps[j].wait()
            g_cps[j] = pltpu.async_copy(
                w_hbm.at[cid_v.at[j]], rows_v.at[j % _NRB], gsem[j % _NRB])

        for j in range(_NRB):
            fire_rows(j)

        out_cps = []
        for j in range(_NCH):
            rb = j % _NRB
            g_cps[j].wait()
            x_cps[j].wait()

            def body(r2, carry):
                for dr in range(2):
                    r = r2 * 2 + dr
                    for gi in range(_D // 16):
                        s = pl.ds(gi * 16, 16)
                        xb_v[j, r, s] = xb_v[j, r, s] + rows_v[rb, r, s]
                return carry

            half = _CHUNK // 2
            lax.fori_loop(0, half // 2, body, 0)
            out_cps.append(pltpu.async_copy(
                xb_v.at[j].at[pl.ds(0, half)],
                out_hbm.at[pl.ds(base + j * _CHUNK, half)], osem))
            lax.fori_loop(half // 2, _CHUNK // 2, body, 0)
            out_cps.append(pltpu.async_copy(
                xb_v.at[j].at[pl.ds(half, half)],
                out_hbm.at[pl.ds(base + j * _CHUNK + half, half)], osem))
            if j + _NRB < _NCH:
                fire_rows(j + _NRB)
        for cp in out_cps:
            cp.wait()

    return k(x, ind2, mapping, weight)


def kernel(x, ind, mapping, weight):
    return _sc_add_gather(x, ind.astype(jnp.int32), mapping.astype(jnp.int32), weight)


# speedup vs baseline: 1.0263x; 1.0192x over previous
import functools

import jax
import jax.numpy as jnp
from jax import lax
from jax.experimental import pallas as pl
from jax.experimental.pallas import tpu as pltpu
from jax.experimental.pallas import tpu_sc as plsc

_BATCH = 16384
_D = 128
_NW = 32
_BPW = _BATCH // _NW
_CHUNK = 128
_NCH = _BPW // _CHUNK
_NRB = 3


def _sc_add_gather(x, ind2, mapping, weight):
    mesh = plsc.VectorSubcoreMesh(core_axis_name="c", subcore_axis_name="s")

    @functools.partial(
        pl.kernel,
        mesh=mesh,
        out_type=jax.ShapeDtypeStruct((_BATCH, _D), jnp.float32),
        scratch_types=[
            pltpu.VMEM((_BPW,), jnp.int32),
            pltpu.VMEM((_NCH, _CHUNK), jnp.int32),
            pltpu.VMEM((_NRB, _CHUNK, _D), jnp.float32),
            pltpu.VMEM((_NCH, _CHUNK, _D), jnp.float32),
        ]
        + [pltpu.SemaphoreType.DMA] * _NCH
        + [pltpu.SemaphoreType.DMA] * _NRB
        + [pltpu.SemaphoreType.DMA] * _NCH
        + [pltpu.SemaphoreType.DMA]
        + [pltpu.SemaphoreType.DMA],
    )
    def k(x_hbm, ind_hbm, map_hbm, w_hbm, out_hbm, idx_v, cid_v, rows_v, xb_v,
          *sems):
        csem = sems[0:_NCH]
        gsem = sems[_NCH:_NCH + _NRB]
        xsem = sems[_NCH + _NRB:2 * _NCH + _NRB]
        osem = sems[2 * _NCH + _NRB]
        isem = sems[2 * _NCH + _NRB + 1]
        wid = lax.axis_index("s") * 2 + lax.axis_index("c")
        base = wid * _BPW
        ind_cp = pltpu.async_copy(ind_hbm.at[pl.ds(base, _BPW)], idx_v, isem)
        x_cps = [
            pltpu.async_copy(
                x_hbm.at[pl.ds(base + j * _CHUNK, _CHUNK)], xb_v.at[j], xsem[j])
            for j in range(_NCH)
        ]
        ind_cp.wait()
        cid_cps = [
            pltpu.async_copy(
                map_hbm.at[idx_v.at[pl.ds(j * _CHUNK, _CHUNK)]],
                cid_v.at[j], csem[j])
            for j in range(_NCH)
        ]

        g_cps = [None] * _NCH

        def fire_rows(j):
            cid_cps[j].wait()
            g_cps[j] = pltpu.async_copy(
                w_hbm.at[cid_v.at[j]], rows_v.at[j % _NRB], gsem[j % _NRB])

        for j in range(_NRB):
            fire_rows(j)

        out_cps = []
        for j in range(_NCH):
            rb = j % _NRB
            g_cps[j].wait()
            x_cps[j].wait()

            def body(r2, carry):
                for dr in range(2):
                    r = r2 * 2 + dr
                    for gi in range(_D // 16):
                        s = pl.ds(gi * 16, 16)
                        xb_v[j, r, s] = xb_v[j, r, s] + rows_v[rb, r, s]
                return carry

            lax.fori_loop(0, _CHUNK // 2, body, 0)
            out_cps.append(pltpu.async_copy(
                xb_v.at[j], out_hbm.at[pl.ds(base + j * _CHUNK, _CHUNK)], osem))
            if j + _NRB < _NCH:
                fire_rows(j + _NRB)
        for cp in out_cps:
            cp.wait()

    return k(x, ind2, mapping, weight)


def kernel(x, ind, mapping, weight):
    return _sc_add_gather(x, ind.astype(jnp.int32), mapping.astype(jnp.int32), weight)


# speedup vs baseline: 1.0421x; 1.0154x over previous
import functools

import jax
import jax.numpy as jnp
from jax import lax
from jax.experimental import pallas as pl
from jax.experimental.pallas import tpu as pltpu
from jax.experimental.pallas import tpu_sc as plsc

_BATCH = 16384
_D = 128
_NW = 32
_BPW = _BATCH // _NW
_CHUNK = 128
_NCH = _BPW // _CHUNK
_NRB = 3


def _sc_add_gather(x, ind2, mapping, weight):
    mesh = plsc.VectorSubcoreMesh(core_axis_name="c", subcore_axis_name="s")

    @functools.partial(
        pl.kernel,
        mesh=mesh,
        out_type=jax.ShapeDtypeStruct((_BATCH, _D), jnp.float32),
        scratch_types=[
            pltpu.VMEM((_BPW,), jnp.int32),
            pltpu.VMEM((_NCH, _CHUNK), jnp.int32),
            pltpu.VMEM((_NRB, _CHUNK, _D), jnp.float32),
            pltpu.VMEM((_BPW, _D), jnp.float32),
        ]
        + [pltpu.SemaphoreType.DMA] * _NCH
        + [pltpu.SemaphoreType.DMA] * _NRB
        + [pltpu.SemaphoreType.DMA]
        + [pltpu.SemaphoreType.DMA]
        + [pltpu.SemaphoreType.DMA],
    )
    def k(x_hbm, ind_hbm, map_hbm, w_hbm, out_hbm, idx_v, cid_v, rows_v, xb_v,
          *sems):
        csem = sems[0:_NCH]
        gsem = sems[_NCH:_NCH + _NRB]
        xsem = sems[_NCH + _NRB]
        osem = sems[_NCH + _NRB + 1]
        isem = sems[_NCH + _NRB + 2]
        wid = lax.axis_index("s") * 2 + lax.axis_index("c")
        base = wid * _BPW
        ind_cp = pltpu.async_copy(ind_hbm.at[pl.ds(base, _BPW)], idx_v, isem)
        x_cp = pltpu.async_copy(x_hbm.at[pl.ds(base, _BPW)], xb_v, xsem)
        ind_cp.wait()
        cid_cps = [
            pltpu.async_copy(
                map_hbm.at[idx_v.at[pl.ds(j * _CHUNK, _CHUNK)]],
                cid_v.at[j], csem[j])
            for j in range(_NCH)
        ]

        g_cps = [None] * _NCH

        def fire_rows(j):
            cid_cps[j].wait()
            g_cps[j] = pltpu.async_copy(
                w_hbm.at[cid_v.at[j]], rows_v.at[j % _NRB], gsem[j % _NRB])

        for j in range(_NRB):
            fire_rows(j)

        out_cps = []
        x_cp.wait()
        for j in range(_NCH):
            rb = j % _NRB
            g_cps[j].wait()

            def body(r2, carry):
                for dr in range(2):
                    r = r2 * 2 + dr
                    for gi in range(_D // 16):
                        s = pl.ds(gi * 16, 16)
                        xb_v[j * _CHUNK + r, s] = (
                            xb_v[j * _CHUNK + r, s] + rows_v[rb, r, s])
                return carry

            lax.fori_loop(0, _CHUNK // 2, body, 0)
            out_cps.append(pltpu.async_copy(
                xb_v.at[pl.ds(j * _CHUNK, _CHUNK)],
                out_hbm.at[pl.ds(base + j * _CHUNK, _CHUNK)], osem))
            if j + _NRB < _NCH:
                fire_rows(j + _NRB)
        for cp in out_cps:
            cp.wait()

    return k(x, ind2, mapping, weight)


def kernel(x, ind, mapping, weight):
    return _sc_add_gather(x, ind.astype(jnp.int32), mapping.astype(jnp.int32), weight)
